# baseline (speedup 1.0000x reference)
import functools
import math

import jax
import jax.numpy as jnp
from jax import lax
from jax.experimental import pallas as pl
from jax.experimental.pallas import tpu as pltpu
from jax.experimental.pallas import tpu_sc as plsc

D_MODEL = 64
SCALE = math.sqrt(D_MODEL)

NUM_CORES = 2
NUM_SUBCORES = 16
NUM_WORKERS = NUM_CORES * NUM_SUBCORES
LANES = 16

TSTRIDE = 133
NBUF = 4


def _transpose_kernel(tt_hbm, ttail_hbm, t3_hbm, ina, tbuf, isems, osems):
    wid = lax.axis_index("s") * NUM_CORES + lax.axis_index("c")
    n_vocab = tt_hbm.shape[1]
    n_full = (n_vocab - 128) // 128 + 1
    max_start = (n_full - 1) * 128
    n_iter = -(-n_full // NUM_WORKERS)

    def col(i):
        return pl.multiple_of(
            jnp.minimum((wid + i * NUM_WORKERS) * 128, max_start), 128
        )

    def load(i, b):
        pltpu.async_copy(tt_hbm.at[:, pl.ds(col(i), 128)], ina[b], isems[b])

    def load_wait(b):
        pltpu.make_async_copy(
            tt_hbm.at[:, pl.ds(0, 128)], ina[b], isems[b]
        ).wait()

    def store(i, b):
        pltpu.async_copy(
            tbuf[b].at[:, pl.ds(0, 128)],
            t3_hbm.at[pl.ds(col(i), 128)],
            osems[b],
        )

    def store_wait(b):
        pltpu.make_async_copy(
            tbuf[b].at[:, pl.ds(0, 128)],
            t3_hbm.at[pl.ds(0, 128)],
            osems[b],
        ).wait()

    def transpose(b):
        @plsc.parallel_loop(0, D_MODEL, step=1, unroll=4)
        def _(d):
            cols = jnp.full((LANES,), d, jnp.int32)
            for j in range(128 // LANES):
                v = ina[b][d, pl.ds(j * LANES, LANES)]
                rows = jax.lax.iota(jnp.int32, LANES) + (j * LANES)
                plsc.store_scatter(tbuf[b], [rows, cols], v)

    load(0, 0)

    def pair(g, _):
        for b in range(2):
            hh = 2 * g + b
            bn = (b + 1) % 2

            @pl.when(hh >= 1)
            def _():
                store_wait(bn)

            load(hh + 1, bn)
            load_wait(b)
            transpose(b)
            store(hh, b)
        return 0

    lax.fori_loop(0, (n_iter - 1) // 2, pair, 0)

    store_wait(1)
    load_wait(0)
    transpose(0)
    store(n_iter - 1, 0)
    store_wait(0)

    tail = n_vocab - n_full * 128
    if tail:
        @pl.when(wid == 0)
        def _():
            pltpu.sync_copy(ttail_hbm, ina[0])
            transpose(0)
            pltpu.sync_copy(
                tbuf[0].at[pl.ds(0, tail), pl.ds(0, 128)],
                t3_hbm.at[pl.ds(n_full * 128, tail)],
            )


def _gather_kernel(x_hbm, t3_hbm, out_hbm, idxblk, idxcol, rows, tbuf,
                   gsems, osems):
    wid = lax.axis_index("s") * NUM_CORES + lax.axis_index("c")
    batch, seq = x_hbm.shape
    b0 = wid * 128

    pltpu.sync_copy(x_hbm.at[pl.ds(b0, 128)], idxblk)

    def build_idx(s, b):
        cols = jnp.full((LANES,), s, jnp.int32)
        for j in range(128 // LANES):
            r = jax.lax.iota(jnp.int32, LANES) + (j * LANES)
            idxcol[b][pl.ds(j * LANES, LANES)] = plsc.load_gather(
                idxblk, [r, cols]
            )

    def gather(b):
        pltpu.async_copy(t3_hbm.at[idxcol[b]], rows[b], gsems[b])

    def gather_wait(b):
        pltpu.make_async_copy(t3_hbm.at[idxcol[b]], rows[b], gsems[b]).wait()

    def writeback(s, b):
        pltpu.async_copy(
            tbuf[b].at[:, :, pl.ds(0, 128)], out_hbm.at[s, :, wid], osems[b]
        )

    def writeback_wait(b):
        pltpu.make_async_copy(
            tbuf[b].at[:, :, pl.ds(0, 128)], out_hbm.at[0, :, wid], osems[b]
        ).wait()

    build_idx(0, 0)
    gather(0)

    def group(g, _):
        for b in range(NBUF):
            s = g * NBUF + b
            b1 = (b + 1) % NBUF

            @pl.when(s >= NBUF - 1)
            def _():
                writeback_wait(b1)

            @pl.when(s + 1 < seq)
            def _():
                build_idx(s + 1, b1)
                gather(b1)

            gather_wait(b)

            @plsc.parallel_loop(0, 128, step=1, unroll=4)
            def _(t):
                cols = jnp.full((LANES,), t, jnp.int32)
                for j in range(D_MODEL // LANES):
                    v = rows[b][t, pl.ds(j * LANES, LANES)] * SCALE
                    d = jax.lax.iota(jnp.int32, LANES) + (j * LANES)
                    plsc.store_scatter(
                        tbuf[b],
                        [jax.lax.shift_right_logical(d, 3), d & 7, cols],
                        v,
                    )

            writeback(s, b)
        return 0

    lax.fori_loop(0, seq // NBUF, group, 0)

    for k in range(1, NBUF):
        writeback_wait((seq - k) % NBUF)


def kernel(x, table):
    batch, seq = x.shape
    n_vocab = table.shape[0]
    tt = table.T
    n_full = (n_vocab - 128) // 128 + 1
    tail = n_vocab - n_full * 128
    ttail = jnp.pad(tt[:, n_full * 128 :], ((0, 0), (0, 128 - tail)))

    mesh = plsc.VectorSubcoreMesh(core_axis_name="c", subcore_axis_name="s")

    run_a = pl.kernel(
        _transpose_kernel,
        out_type=jax.ShapeDtypeStruct((n_vocab, 128), jnp.float32),
        mesh=mesh,
        scratch_types=[
            [pltpu.VMEM((D_MODEL, 128), jnp.float32) for _ in range(2)],
            [pltpu.VMEM((128, TSTRIDE), jnp.float32) for _ in range(2)],
            [pltpu.SemaphoreType.DMA for _ in range(2)],
            [pltpu.SemaphoreType.DMA for _ in range(2)],
        ],
        compiler_params=pltpu.CompilerParams(
            use_tc_tiling_on_sc=True, needs_layout_passes=False
        ),
    )

    run_b = pl.kernel(
        _gather_kernel,
        out_type=jax.ShapeDtypeStruct((seq, 8, NUM_WORKERS, 8, 128), jnp.float32),
        mesh=mesh,
        scratch_types=[
            pltpu.VMEM((128, seq), jnp.int32),
            [pltpu.VMEM((128,), jnp.int32) for _ in range(NBUF)],
            [pltpu.VMEM((128, 128), jnp.float32) for _ in range(NBUF)],
            [pltpu.VMEM((8, 8, 129), jnp.float32) for _ in range(NBUF)],
            [pltpu.SemaphoreType.DMA for _ in range(NBUF)],
            [pltpu.SemaphoreType.DMA for _ in range(NBUF)],
        ],
        compiler_params=pltpu.CompilerParams(
            use_tc_tiling_on_sc=False, needs_layout_passes=False
        ),
    )

    t3 = run_a(tt, ttail)
    out5 = run_b(x, t3)
    return out5.transpose(2, 4, 0, 1, 3).reshape(batch, seq, D_MODEL)


# speedup vs baseline: 1.0036x; 1.0036x over previous
import functools
import math

import jax
import jax.numpy as jnp
from jax import lax
from jax.experimental import pallas as pl
from jax.experimental.pallas import tpu as pltpu
from jax.experimental.pallas import tpu_sc as plsc

D_MODEL = 64
SCALE = math.sqrt(D_MODEL)

NUM_CORES = 2
NUM_SUBCORES = 16
NUM_WORKERS = NUM_CORES * NUM_SUBCORES
LANES = 16

TSTRIDE = 133
NBUF = 4


def _transpose_kernel(tt_hbm, ttail_hbm, t3_hbm, ina, tbuf, isems, osems):
    wid = lax.axis_index("s") * NUM_CORES + lax.axis_index("c")
    n_vocab = tt_hbm.shape[1]
    width = ina[0].shape[1]
    n_full = (n_vocab - width) // width + 1
    max_start = (n_full - 1) * width
    n_iter = -(-n_full // NUM_WORKERS)

    def col(i):
        return pl.multiple_of(
            jnp.minimum((wid + i * NUM_WORKERS) * width, max_start), width
        )

    def load(i, b):
        pltpu.async_copy(tt_hbm.at[:, pl.ds(col(i), width)], ina[b], isems[b])

    def load_wait(b):
        pltpu.make_async_copy(
            tt_hbm.at[:, pl.ds(0, width)], ina[b], isems[b]
        ).wait()

    def store(i, b):
        pltpu.async_copy(
            tbuf[b].at[:, pl.ds(0, 128)],
            t3_hbm.at[pl.ds(col(i), width)],
            osems[b],
        )

    def store_wait(b):
        pltpu.make_async_copy(
            tbuf[b].at[:, pl.ds(0, 128)],
            t3_hbm.at[pl.ds(0, width)],
            osems[b],
        ).wait()

    rows_list = [
        jax.lax.iota(jnp.int32, LANES) + (j * LANES)
        for j in range(width // LANES)
    ]

    def transpose(b):
        @plsc.parallel_loop(0, D_MODEL, step=1, unroll=8)
        def _(d):
            cols = jnp.full((LANES,), d, jnp.int32)
            for j in range(width // LANES):
                v = ina[b][d, pl.ds(j * LANES, LANES)]
                plsc.store_scatter(tbuf[b], [rows_list[j], cols], v)

    load(0, 0)

    def pair(g, _):
        for b in range(2):
            hh = 2 * g + b
            bn = (b + 1) % 2

            @pl.when(hh >= 1)
            def _():
                store_wait(bn)

            load(hh + 1, bn)
            load_wait(b)
            transpose(b)
            store(hh, b)
        return 0

    lax.fori_loop(0, (n_iter - 1) // 2, pair, 0)

    store_wait(1)
    load_wait(0)
    transpose(0)
    store(n_iter - 1, 0)
    store_wait(0)

    tail = n_vocab - n_full * width
    if tail:
        @pl.when(wid == 0)
        def _():
            pltpu.sync_copy(ttail_hbm, ina[0].at[:, pl.ds(0, 128)])

            @plsc.parallel_loop(0, D_MODEL, step=1, unroll=4)
            def _(d):
                cols = jnp.full((LANES,), d, jnp.int32)
                for j in range(tail // LANES):
                    v = ina[0][d, pl.ds(j * LANES, LANES)]
                    rows = jax.lax.iota(jnp.int32, LANES) + (j * LANES)
                    plsc.store_scatter(tbuf[0], [rows, cols], v)

            pltpu.sync_copy(
                tbuf[0].at[pl.ds(0, tail), pl.ds(0, 128)],
                t3_hbm.at[pl.ds(n_full * width, tail)],
            )


def _gather_kernel(x_hbm, t3_hbm, out_hbm, idxblk, idxcol, rows, tbuf,
                   gsems, osems):
    wid = lax.axis_index("s") * NUM_CORES + lax.axis_index("c")
    batch, seq = x_hbm.shape
    b0 = wid * 128

    pltpu.sync_copy(x_hbm.at[pl.ds(b0, 128)], idxblk)

    def build_idx(s, b):
        cols = jnp.full((LANES,), s, jnp.int32)
        for j in range(128 // LANES):
            r = jax.lax.iota(jnp.int32, LANES) + (j * LANES)
            idxcol[b][pl.ds(j * LANES, LANES)] = plsc.load_gather(
                idxblk, [r, cols]
            )

    def gather(b):
        pltpu.async_copy(t3_hbm.at[idxcol[b]], rows[b], gsems[b])

    def gather_wait(b):
        pltpu.make_async_copy(t3_hbm.at[idxcol[b]], rows[b], gsems[b]).wait()

    def writeback(s, b):
        pltpu.async_copy(
            tbuf[b].at[:, :, pl.ds(0, 128)], out_hbm.at[s, :, wid], osems[b]
        )

    def writeback_wait(b):
        pltpu.make_async_copy(
            tbuf[b].at[:, :, pl.ds(0, 128)], out_hbm.at[0, :, wid], osems[b]
        ).wait()

    build_idx(0, 0)
    gather(0)

    def group(g, _):
        for b in range(NBUF):
            s = g * NBUF + b
            b1 = (b + 1) % NBUF

            @pl.when(s >= NBUF - 1)
            def _():
                writeback_wait(b1)

            @pl.when(s + 1 < seq)
            def _():
                build_idx(s + 1, b1)
                gather(b1)

            gather_wait(b)

            @plsc.parallel_loop(0, 128, step=1, unroll=4)
            def _(t):
                cols = jnp.full((LANES,), t, jnp.int32)
                for j in range(D_MODEL // LANES):
                    v = rows[b][t, pl.ds(j * LANES, LANES)] * SCALE
                    d = jax.lax.iota(jnp.int32, LANES) + (j * LANES)
                    plsc.store_scatter(
                        tbuf[b],
                        [jax.lax.shift_right_logical(d, 3), d & 7, cols],
                        v,
                    )

            writeback(s, b)
        return 0

    lax.fori_loop(0, seq // NBUF, group, 0)

    for k in range(1, NBUF):
        writeback_wait((seq - k) % NBUF)


def kernel(x, table):
    batch, seq = x.shape
    n_vocab = table.shape[0]
    tt = table.T
    n_full = (n_vocab - 128) // 128 + 1
    tail = n_vocab - n_full * 128
    ttail = jnp.pad(tt[:, n_full * 128 :], ((0, 0), (0, 128 - tail)))

    mesh = plsc.VectorSubcoreMesh(core_axis_name="c", subcore_axis_name="s")

    run_a = pl.kernel(
        _transpose_kernel,
        out_type=jax.ShapeDtypeStruct((n_vocab, 128), jnp.float32),
        mesh=mesh,
        scratch_types=[
            [pltpu.VMEM((D_MODEL, 128), jnp.float32) for _ in range(2)],
            [pltpu.VMEM((128, TSTRIDE), jnp.float32) for _ in range(2)],
            [pltpu.SemaphoreType.DMA for _ in range(2)],
            [pltpu.SemaphoreType.DMA for _ in range(2)],
        ],
        compiler_params=pltpu.CompilerParams(
            use_tc_tiling_on_sc=True, needs_layout_passes=False
        ),
    )

    run_b = pl.kernel(
        _gather_kernel,
        out_type=jax.ShapeDtypeStruct((seq, 8, NUM_WORKERS, 8, 128), jnp.float32),
        mesh=mesh,
        scratch_types=[
            pltpu.VMEM((128, seq), jnp.int32),
            [pltpu.VMEM((128,), jnp.int32) for _ in range(NBUF)],
            [pltpu.VMEM((128, 128), jnp.float32) for _ in range(NBUF)],
            [pltpu.VMEM((8, 8, 129), jnp.float32) for _ in range(NBUF)],
            [pltpu.SemaphoreType.DMA for _ in range(NBUF)],
            [pltpu.SemaphoreType.DMA for _ in range(NBUF)],
        ],
        compiler_params=pltpu.CompilerParams(
            use_tc_tiling_on_sc=False, needs_layout_passes=False
        ),
    )

    t3 = run_a(tt, ttail)
    out5 = run_b(x, t3)
    return out5.transpose(2, 4, 0, 1, 3).reshape(batch, seq, D_MODEL)


# speedup vs baseline: 1.6131x; 1.6073x over previous
import functools
import math

import jax
import jax.numpy as jnp
from jax import lax
from jax.experimental import pallas as pl
from jax.experimental.pallas import tpu as pltpu
from jax.experimental.pallas import tpu_sc as plsc

D_MODEL = 64
SCALE = math.sqrt(D_MODEL)

NUM_CORES = 2
NUM_SUBCORES = 16
NUM_WORKERS = NUM_CORES * NUM_SUBCORES
LANES = 16

TSTRIDE = 133
NBUF = 4


def _transpose_kernel(tt_hbm, ttail_hbm, t3_hbm, ina, tbuf, isems, osems):
    wid = lax.axis_index("s") * NUM_CORES + lax.axis_index("c")
    n_vocab = tt_hbm.shape[1]
    width = ina[0].shape[1]
    n_full = (n_vocab - width) // width + 1
    max_start = (n_full - 1) * width
    n_iter = -(-n_full // NUM_WORKERS)

    def col(i):
        return pl.multiple_of(
            jnp.minimum((wid + i * NUM_WORKERS) * width, max_start), width
        )

    def load(i, b):
        pltpu.async_copy(tt_hbm.at[:, pl.ds(col(i), width)], ina[b], isems[b])

    def load_wait(b):
        pltpu.make_async_copy(
            tt_hbm.at[:, pl.ds(0, width)], ina[b], isems[b]
        ).wait()

    def store(i, b):
        pltpu.async_copy(
            tbuf[b].at[:, pl.ds(0, 128)],
            t3_hbm.at[pl.ds(col(i), width)],
            osems[b],
        )

    def store_wait(b):
        pltpu.make_async_copy(
            tbuf[b].at[:, pl.ds(0, 128)],
            t3_hbm.at[pl.ds(0, width)],
            osems[b],
        ).wait()

    rows_list = [
        jax.lax.iota(jnp.int32, LANES) + (j * LANES)
        for j in range(width // LANES)
    ]

    def transpose(b):
        @plsc.parallel_loop(0, D_MODEL, step=1, unroll=8)
        def _(d):
            cols = jnp.full((LANES,), d, jnp.int32)
            for j in range(width // LANES):
                v = ina[b][d, pl.ds(j * LANES, LANES)]
                plsc.store_scatter(tbuf[b], [rows_list[j], cols], v)

    load(0, 0)

    def pair(g, _):
        for b in range(2):
            hh = 2 * g + b
            bn = (b + 1) % 2

            @pl.when(hh >= 1)
            def _():
                store_wait(bn)

            load(hh + 1, bn)
            load_wait(b)
            transpose(b)
            store(hh, b)
        return 0

    lax.fori_loop(0, (n_iter - 1) // 2, pair, 0)

    store_wait(1)
    load_wait(0)
    transpose(0)
    store(n_iter - 1, 0)
    store_wait(0)

    tail = n_vocab - n_full * width
    if tail:
        @pl.when(wid == 0)
        def _():
            pltpu.sync_copy(ttail_hbm, ina[0].at[:, pl.ds(0, 128)])

            @plsc.parallel_loop(0, D_MODEL, step=1, unroll=4)
            def _(d):
                cols = jnp.full((LANES,), d, jnp.int32)
                for j in range(tail // LANES):
                    v = ina[0][d, pl.ds(j * LANES, LANES)]
                    rows = jax.lax.iota(jnp.int32, LANES) + (j * LANES)
                    plsc.store_scatter(tbuf[0], [rows, cols], v)

            pltpu.sync_copy(
                tbuf[0].at[pl.ds(0, tail), pl.ds(0, 128)],
                t3_hbm.at[pl.ds(n_full * width, tail)],
            )


def _gather_kernel(x_hbm, t3_hbm, out_hbm, idxblk, idxcol, rows, tbuf,
                   gsems, osems):
    wid = lax.axis_index("s") * NUM_CORES + lax.axis_index("c")
    batch, seq = x_hbm.shape
    b0 = wid * 128

    pltpu.sync_copy(x_hbm.at[pl.ds(b0, 128)], idxblk)

    def build_idx(s, b):
        cols = jnp.full((LANES,), s, jnp.int32)
        for j in range(128 // LANES):
            r = jax.lax.iota(jnp.int32, LANES) + (j * LANES)
            idxcol[b][pl.ds(j * LANES, LANES)] = plsc.load_gather(
                idxblk, [r, cols]
            )

    def gather(b):
        pltpu.async_copy(t3_hbm.at[idxcol[b]], rows[b], gsems[b])

    def gather_wait(b):
        pltpu.make_async_copy(t3_hbm.at[idxcol[b]], rows[b], gsems[b]).wait()

    def writeback(s, b):
        pltpu.async_copy(
            tbuf[b].at[:, :, pl.ds(0, 128)], out_hbm.at[s, :, wid], osems[b]
        )

    def writeback_wait(b):
        pltpu.make_async_copy(
            tbuf[b].at[:, :, pl.ds(0, 128)], out_hbm.at[0, :, wid], osems[b]
        ).wait()

    build_idx(0, 0)
    gather(0)

    def group(g, _):
        for b in range(NBUF):
            s = g * NBUF + b
            b1 = (b + 1) % NBUF

            @pl.when(s >= NBUF - 1)
            def _():
                writeback_wait(b1)

            @pl.when(s + 1 < seq)
            def _():
                build_idx(s + 1, b1)
                gather(b1)

            gather_wait(b)

            @plsc.parallel_loop(0, 128, step=1, unroll=4)
            def _(t):
                cols = jnp.full((LANES,), t, jnp.int32)
                for j in range(D_MODEL // LANES):
                    v = rows[b][t, pl.ds(j * LANES, LANES)] * SCALE
                    d = jax.lax.iota(jnp.int32, LANES) + (j * LANES)
                    plsc.store_scatter(
                        tbuf[b],
                        [jax.lax.shift_right_logical(d, 3), d & 7, cols],
                        v,
                    )

            writeback(s, b)
        return 0

    lax.fori_loop(0, seq // NBUF, group, 0)

    for k in range(1, NBUF):
        writeback_wait((seq - k) % NBUF)


def kernel(x, table):
    batch, seq = x.shape
    n_vocab = table.shape[0]
    tt = table.T
    n_full = (n_vocab - 128) // 128 + 1
    tail = n_vocab - n_full * 128
    ttail = jnp.pad(tt[:, n_full * 128 :], ((0, 0), (0, 128 - tail)))

    mesh = plsc.VectorSubcoreMesh(core_axis_name="c", subcore_axis_name="s")

    run_a = pl.kernel(
        _transpose_kernel,
        out_type=jax.ShapeDtypeStruct((n_vocab, 128), jnp.float32),
        mesh=mesh,
        scratch_types=[
            [pltpu.VMEM((D_MODEL, 128), jnp.float32) for _ in range(2)],
            [pltpu.VMEM((128, TSTRIDE), jnp.float32) for _ in range(2)],
            [pltpu.SemaphoreType.DMA for _ in range(2)],
            [pltpu.SemaphoreType.DMA for _ in range(2)],
        ],
        compiler_params=pltpu.CompilerParams(
            use_tc_tiling_on_sc=True, needs_layout_passes=False
        ),
    )

    run_b = pl.kernel(
        _gather_kernel,
        out_type=jax.ShapeDtypeStruct((seq, 8, NUM_WORKERS, 8, 128), jnp.float32),
        mesh=mesh,
        scratch_types=[
            pltpu.VMEM((128, seq), jnp.int32),
            [pltpu.VMEM((128,), jnp.int32) for _ in range(NBUF)],
            [pltpu.VMEM((128, D_MODEL), jnp.float32) for _ in range(NBUF)],
            [pltpu.VMEM((8, 8, 129), jnp.float32) for _ in range(NBUF)],
            [pltpu.SemaphoreType.DMA for _ in range(NBUF)],
            [pltpu.SemaphoreType.DMA for _ in range(NBUF)],
        ],
        compiler_params=pltpu.CompilerParams(
            use_tc_tiling_on_sc=False, needs_layout_passes=False
        ),
    )

    out5 = run_b(x, table)
    return out5.transpose(2, 4, 0, 1, 3).reshape(batch, seq, D_MODEL)


# speedup vs baseline: 1.6213x; 1.0051x over previous
import math

import jax
import jax.numpy as jnp
from jax import lax
from jax.experimental import pallas as pl
from jax.experimental.pallas import tpu as pltpu
from jax.experimental.pallas import tpu_sc as plsc

D_MODEL = 64
SCALE = math.sqrt(D_MODEL)

NUM_CORES = 2
NUM_SUBCORES = 16
NUM_WORKERS = NUM_CORES * NUM_SUBCORES
LANES = 16

NBUF = 4


def _gather_kernel(x_hbm, t3_hbm, out_hbm, idxblk, idxcol, rows, tbuf,
                   gsems, osems):
    wid = lax.axis_index("s") * NUM_CORES + lax.axis_index("c")
    batch, seq = x_hbm.shape
    b0 = wid * 128

    pltpu.sync_copy(x_hbm.at[pl.ds(b0, 128)], idxblk)

    def build_idx(s, b):
        cols = jnp.full((LANES,), s, jnp.int32)
        for j in range(128 // LANES):
            r = jax.lax.iota(jnp.int32, LANES) + (j * LANES)
            idxcol[b][pl.ds(j * LANES, LANES)] = plsc.load_gather(
                idxblk, [r, cols]
            )

    def gather(b):
        pltpu.async_copy(t3_hbm.at[idxcol[b]], rows[b], gsems[b])

    def gather_wait(b):
        pltpu.make_async_copy(t3_hbm.at[idxcol[b]], rows[b], gsems[b]).wait()

    def writeback(s, b):
        pltpu.async_copy(
            tbuf[b].at[:, :, pl.ds(0, 128)], out_hbm.at[s, :, wid], osems[b]
        )

    def writeback_wait(b):
        pltpu.make_async_copy(
            tbuf[b].at[:, :, pl.ds(0, 128)], out_hbm.at[0, :, wid], osems[b]
        ).wait()

    build_idx(0, 0)
    gather(0)

    def group(g, _):
        for b in range(NBUF):
            s = g * NBUF + b
            b1 = (b + 1) % NBUF

            @pl.when(s >= NBUF - 1)
            def _():
                writeback_wait(b1)

            @pl.when(s + 1 < seq)
            def _():
                build_idx(s + 1, b1)
                gather(b1)

            gather_wait(b)

            @plsc.parallel_loop(0, 128, step=1, unroll=4)
            def _(t):
                cols = jnp.full((LANES,), t, jnp.int32)
                for j in range(D_MODEL // LANES):
                    v = rows[b][t, pl.ds(j * LANES, LANES)] * SCALE
                    d = jax.lax.iota(jnp.int32, LANES) + (j * LANES)
                    plsc.store_scatter(
                        tbuf[b],
                        [jax.lax.shift_right_logical(d, 3), d & 7, cols],
                        v,
                    )

            writeback(s, b)
        return 0

    lax.fori_loop(0, seq // NBUF, group, 0)

    for k in range(1, NBUF):
        writeback_wait((seq - k) % NBUF)


def kernel(x, table):
    batch, seq = x.shape

    mesh = plsc.VectorSubcoreMesh(core_axis_name="c", subcore_axis_name="s")
    run = pl.kernel(
        _gather_kernel,
        out_type=jax.ShapeDtypeStruct((seq, 8, NUM_WORKERS, 8, 128), jnp.float32),
        mesh=mesh,
        scratch_types=[
            pltpu.VMEM((128, seq), jnp.int32),
            [pltpu.VMEM((128,), jnp.int32) for _ in range(NBUF)],
            [pltpu.VMEM((128, D_MODEL), jnp.float32) for _ in range(NBUF)],
            [pltpu.VMEM((8, 8, 129), jnp.float32) for _ in range(NBUF)],
            [pltpu.SemaphoreType.DMA for _ in range(NBUF)],
            [pltpu.SemaphoreType.DMA for _ in range(NBUF)],
        ],
        compiler_params=pltpu.CompilerParams(
            use_tc_tiling_on_sc=False, needs_layout_passes=False
        ),
    )

    out5 = run(x, table)
    return out5.transpose(2, 4, 0, 1, 3).reshape(batch, seq, D_MODEL)
